# baseline (speedup 1.0000x reference)
import jax
import jax.numpy as jnp
from jax.experimental import pallas as pl


def _gcn(x, W, b, src, dst, ew):
    n = x.shape[0]
    loop = jnp.arange(n)
    src2 = jnp.concatenate([src, loop])
    dst2 = jnp.concatenate([dst, loop])
    ew2 = jnp.concatenate([ew, jnp.ones((n,), x.dtype)])
    deg = jnp.zeros((n,), x.dtype).at[dst2].add(ew2)
    dinv = jnp.where(deg > 0, jax.lax.rsqrt(jnp.maximum(deg, 1e-12)), 0.0)
    norm = dinv[src2] * ew2 * dinv[dst2]
    h = x @ W
    out = jnp.zeros((n, W.shape[1]), x.dtype).at[dst2].add(h[src2] * norm[:, None])
    return out + b


def _bn(x, g, be):
    m = jnp.mean(x, axis=0)
    v = jnp.var(x, axis=0)
    return (x - m) * jax.lax.rsqrt(v + 1e-5) * g + be


def _fc_kernel(p_ref, w_ref, b_ref, o_ref):
    o_ref[...] = p_ref[...] @ w_ref[...] + b_ref[...]


def kernel(x, edge_index, edge_attr, batch, W1, b1, g1, be1, W2, b2, g2, be2, W3, b3, g3, be3, Wfc, bfc):
    src, dst = edge_index[0], edge_index[1]
    h = jax.nn.relu(_bn(_gcn(x, W1, b1, src, dst, edge_attr), g1, be1))
    h = jax.nn.relu(_bn(_gcn(h, W2, b2, src, dst, edge_attr), g2, be2))
    res = h
    h = jax.nn.relu(_bn(_gcn(h, W3, b3, src, dst, edge_attr), g3, be3))
    h = h + res
    pooled = jax.ops.segment_sum(h, batch, num_segments=64)
    counts = jax.ops.segment_sum(jnp.ones((h.shape[0], 1), h.dtype), batch, num_segments=64)
    pooled = pooled / jnp.maximum(counts, 1.0)
    return pl.pallas_call(
        _fc_kernel,
        out_shape=jax.ShapeDtypeStruct((pooled.shape[0], Wfc.shape[1]), pooled.dtype),
    )(pooled, Wfc, bfc)


# speedup vs baseline: 5.3780x; 5.3780x over previous
import dataclasses
import functools

import jax
import jax.numpy as jnp
from jax import lax
from jax.experimental import pallas as pl
from jax.experimental.pallas import tpu as pltpu
from jax.experimental.pallas import tpu_sc as plsc

_N = 100000
_E = 1600000
_G = 64
_NP = 100352
_NB = 98
_BR = 1024
_R2 = _NP // 128
_NSUB = 16
_CH = _NP // _NSUB
_EPADW = 100352
_EPAD = _EPADW * _NSUB

_f32 = jnp.float32
_i32 = jnp.int32


def _sc_mesh():
    return plsc.VectorSubcoreMesh(core_axis_name="c", subcore_axis_name="s")


def _sc_params():
    cp = pltpu.CompilerParams()
    fields = pltpu.CompilerParams.__dataclass_fields__
    if "needs_layout_passes" in fields:
        cp = dataclasses.replace(cp, needs_layout_passes=False)
    if "use_tc_tiling_on_sc" in fields:
        cp = dataclasses.replace(cp, use_tc_tiling_on_sc=False)
    return cp


def _sc_deg(dst, ew, zrow):
    WE = 5000
    EPW = _E // 32
    NWIN = EPW // WE

    @functools.partial(
        pl.kernel,
        out_type=jax.ShapeDtypeStruct((2, _NP), _f32),
        mesh=_sc_mesh(),
        compiler_params=_sc_params(),
        scratch_types=[
            pltpu.VMEM((WE,), _i32),
            pltpu.VMEM((WE,), _f32),
            pltpu.VMEM_SHARED((_NP,), _f32),
        ],
    )
    def k(dst_hbm, ew_hbm, z_hbm, out_hbm, dst_v, ew_v, acc_sh):
        cid = lax.axis_index("c")
        sid = lax.axis_index("s")
        pltpu.sync_copy(z_hbm, acc_sh.at[pl.ds(sid * _CH, _CH)])
        plsc.subcore_barrier()
        base = (cid * _NSUB + sid) * EPW

        @pl.loop(0, NWIN)
        def _(w):
            eb = base + w * WE
            pltpu.sync_copy(dst_hbm.at[pl.ds(eb, WE)], dst_v)
            pltpu.sync_copy(ew_hbm.at[pl.ds(eb, WE)], ew_v)
            pltpu.sync_copy(ew_v, acc_sh.at[dst_v], add=True)

        plsc.subcore_barrier()
        for c in range(2):
            @pl.when(cid == c)
            def _(c=c):
                pltpu.sync_copy(acc_sh.at[pl.ds(sid * _CH, _CH)],
                                out_hbm.at[c].at[pl.ds(sid * _CH, _CH)])

    return k(dst, ew, zrow)


def _sc_sx(src, dst, ew, xp, zrow):
    WE = 5000
    EPW = _E // 32
    NWIN = EPW // WE

    @functools.partial(
        pl.kernel,
        out_type=jax.ShapeDtypeStruct((2, _NP), _f32),
        mesh=_sc_mesh(),
        compiler_params=_sc_params(),
        scratch_types=[
            pltpu.VMEM((WE,), _i32),
            pltpu.VMEM((WE,), _i32),
            pltpu.VMEM((WE,), _f32),
            pltpu.VMEM((WE,), _f32),
            pltpu.VMEM((_NP,), _f32),
            pltpu.VMEM_SHARED((_NP,), _f32),
        ],
    )
    def k(src_hbm, dst_hbm, ew_hbm, xp_hbm, z_hbm, out_hbm,
          src_v, dst_v, ew_v, vals_v, xp_v, acc_sh):
        cid = lax.axis_index("c")
        sid = lax.axis_index("s")
        pltpu.sync_copy(z_hbm, acc_sh.at[pl.ds(sid * _CH, _CH)])
        pltpu.sync_copy(xp_hbm, xp_v)
        plsc.subcore_barrier()
        base = (cid * _NSUB + sid) * EPW

        @pl.loop(0, NWIN)
        def _(w):
            eb = base + w * WE
            pltpu.sync_copy(src_hbm.at[pl.ds(eb, WE)], src_v)
            pltpu.sync_copy(dst_hbm.at[pl.ds(eb, WE)], dst_v)
            pltpu.sync_copy(ew_hbm.at[pl.ds(eb, WE)], ew_v)

            @pl.loop(0, WE, step=16)
            def _(j):
                s16 = src_v[pl.ds(j, 16)]
                v16 = plsc.load_gather(xp_v, [s16])
                vals_v[pl.ds(j, 16)] = v16 * ew_v[pl.ds(j, 16)]

            pltpu.sync_copy(vals_v, acc_sh.at[dst_v], add=True)

        plsc.subcore_barrier()
        for c in range(2):
            @pl.when(cid == c)
            def _(c=c):
                pltpu.sync_copy(acc_sh.at[pl.ds(sid * _CH, _CH)],
                                out_hbm.at[c].at[pl.ds(sid * _CH, _CH)])

    return k(src, dst, ew, xp, zrow)


def _sc_agg(table, src, dst, ew, nslab, zrow16):
    WE = 1024
    NCH = 8
    EPW = _EPADW
    NWIN = EPW // WE
    RB = EPW // 128

    @functools.partial(
        pl.kernel,
        out_type=jax.ShapeDtypeStruct((nslab, _NP, 16), _f32),
        mesh=_sc_mesh(),
        compiler_params=_sc_params(),
        scratch_types=[
            pltpu.VMEM((NCH, 128), _i32),
            pltpu.VMEM((NCH, 128), _i32),
            pltpu.VMEM((NCH, 128), _i32),
            pltpu.VMEM((WE,), _f32),
            pltpu.VMEM((WE, 16), _f32),
            pltpu.VMEM_SHARED((_NP, 16), _f32),
        ],
    )
    def k(tab_hbm, src2_hbm, dst2_hbm, ew_hbm, z_hbm, out_hbm,
          src_v, idx_v, dst_v, ew_v, rows_v, acc_sh):
        cid = lax.axis_index("c")
        sid = lax.axis_index("s")
        for k in range(nslab // 2):
            s = k * 2 + cid
            pltpu.sync_copy(z_hbm, acc_sh.at[pl.ds(sid * _CH, _CH)])
            plsc.subcore_barrier()

            @pl.loop(0, NWIN)
            def _(w):
                rb = sid * RB + w * NCH
                pltpu.sync_copy(src2_hbm.at[pl.ds(rb, NCH)], src_v)
                pltpu.sync_copy(dst2_hbm.at[pl.ds(rb, NCH)], dst_v)
                pltpu.sync_copy(ew_hbm.at[pl.ds(sid * EPW + w * WE, WE)],
                                ew_v)

                for g in range(NCH):
                    @pl.loop(0, 128, step=16)
                    def _(j, g=g):
                        s16 = src_v[g, pl.ds(j, 16)]
                        idx_v[g, pl.ds(j, 16)] = s16 * nslab + s

                for g in range(NCH):
                    pltpu.sync_copy(tab_hbm.at[idx_v.at[g]],
                                    rows_v.at[pl.ds(g * 128, 128)])

                @pl.loop(0, WE, step=4)
                def _(j):
                    for u in range(4):
                        bc = plsc.load_gather(
                            ew_v, [jnp.full((16,), j + u, _i32)])
                        rows_v[j + u] = rows_v[j + u] * bc

                for g in range(NCH):
                    pltpu.sync_copy(rows_v.at[pl.ds(g * 128, 128)],
                                    acc_sh.at[dst_v.at[g]], add=True)

            plsc.subcore_barrier()
            pltpu.sync_copy(acc_sh.at[pl.ds(sid * _CH, _CH)],
                            out_hbm.at[s].at[pl.ds(sid * _CH, _CH)])
            plsc.subcore_barrier()

    return k(table, src, dst, ew, zrow16)


def _tc_prep(degp, x2d):
    def body(d_ref, x_ref, dinv_ref, xp_ref):
        deg = d_ref[0] + d_ref[1] + 1.0
        dinv = lax.rsqrt(deg)
        dinv_ref[...] = dinv
        xp_ref[...] = x_ref[...] * dinv

    return pl.pallas_call(
        body,
        out_shape=[jax.ShapeDtypeStruct((_R2, 128), _f32)] * 2,
    )(degp, x2d)


def _tc_sstats(sxp, xp2d, dinv2d):
    def body(p_ref, xp_ref, di_ref, s_ref, mu_ref, var_ref):
        sarr = di_ref[...] * (p_ref[0] + p_ref[1] + xp_ref[...])
        s_ref[...] = sarr
        mu = jnp.sum(sarr) / _N
        var = jnp.sum(sarr * sarr) / _N - mu * mu
        mu_ref[...] = jnp.full((8, 128), mu)
        var_ref[...] = jnp.full((8, 128), var)

    return pl.pallas_call(
        body,
        out_shape=[jax.ShapeDtypeStruct((_R2, 128), _f32),
                   jax.ShapeDtypeStruct((8, 128), _f32),
                   jax.ShapeDtypeStruct((8, 128), _f32)],
    )(sxp, xp2d, dinv2d)


def _tc_affine(s_col, dinv_col, a1, c1):
    def body(s_ref, d_ref, a_ref, c_ref, o_ref):
        o_ref[...] = d_ref[...] * jnp.maximum(
            s_ref[...] * a_ref[...] + c_ref[...], 0.0)

    return pl.pallas_call(
        body,
        grid=(_NB,),
        in_specs=[
            pl.BlockSpec((_BR, 1), lambda i: (i, 0)),
            pl.BlockSpec((_BR, 1), lambda i: (i, 0)),
            pl.BlockSpec((1, 64), lambda i: (0, 0)),
            pl.BlockSpec((1, 64), lambda i: (0, 0)),
        ],
        out_specs=pl.BlockSpec((_BR, 64), lambda i: (i, 0)),
        out_shape=jax.ShapeDtypeStruct((_NP, 64), _f32),
    )(s_col, dinv_col, a1, c1)


def _tc_mm_stats(aggt, zp, dinv_col, Wmat, brow):
    din, dout = Wmat.shape

    def body(agg_ref, zp_ref, d_ref, w_ref, b_ref, t_ref, ps_ref, pq_ref):
        i = pl.program_id(0)
        u = d_ref[...] * (agg_ref[...] + zp_ref[...])
        t = jnp.dot(u, w_ref[...], preferred_element_type=_f32) + b_ref[...]
        t_ref[...] = t
        rid = i * _BR + lax.broadcasted_iota(_i32, (_BR, 1), 0)
        tm = t * (rid < _N).astype(_f32)
        ps_ref[...] = jnp.sum(tm, axis=0, keepdims=True)[None]
        pq_ref[...] = jnp.sum(tm * tm, axis=0, keepdims=True)[None]

    return pl.pallas_call(
        body,
        grid=(_NB,),
        in_specs=[
            pl.BlockSpec((_BR, din), lambda i: (i, 0)),
            pl.BlockSpec((_BR, din), lambda i: (i, 0)),
            pl.BlockSpec((_BR, 1), lambda i: (i, 0)),
            pl.BlockSpec((din, dout), lambda i: (0, 0)),
            pl.BlockSpec((1, dout), lambda i: (0, 0)),
        ],
        out_specs=[
            pl.BlockSpec((_BR, dout), lambda i: (i, 0)),
            pl.BlockSpec((1, 1, dout), lambda i: (i, 0, 0)),
            pl.BlockSpec((1, 1, dout), lambda i: (i, 0, 0)),
        ],
        out_shape=[jax.ShapeDtypeStruct((_NP, dout), _f32),
                   jax.ShapeDtypeStruct((_NB, 1, dout), _f32),
                   jax.ShapeDtypeStruct((_NB, 1, dout), _f32)],
    )(aggt, zp, dinv_col, Wmat, brow)


def _tc_bnparams(ps, pq, grow, berow):
    dout = ps.shape[-1]

    def body(ps_ref, pq_ref, g_ref, b_ref, a_ref, c_ref):
        ssum = jnp.sum(ps_ref[...], axis=(0, 1))
        sq = jnp.sum(pq_ref[...], axis=(0, 1))
        mu = ssum / _N
        var = sq / _N - mu * mu
        a = lax.rsqrt(var + 1e-5) * g_ref[0]
        c = b_ref[0] - mu * a
        a_ref[...] = a[None, :]
        c_ref[...] = c[None, :]

    return pl.pallas_call(
        body,
        out_shape=[jax.ShapeDtypeStruct((1, dout), _f32)] * 2,
    )(ps, pq, grow, berow)


def _tc_bnrelu(t2, dinv_col, a2, c2):
    def body(t_ref, d_ref, a_ref, c_ref, z_ref, zp_ref):
        z = jnp.maximum(t_ref[...] * a_ref[...] + c_ref[...], 0.0)
        z_ref[...] = z
        zp_ref[...] = d_ref[...] * z

    return pl.pallas_call(
        body,
        grid=(_NB,),
        in_specs=[
            pl.BlockSpec((_BR, 128), lambda i: (i, 0)),
            pl.BlockSpec((_BR, 1), lambda i: (i, 0)),
            pl.BlockSpec((1, 128), lambda i: (0, 0)),
            pl.BlockSpec((1, 128), lambda i: (0, 0)),
        ],
        out_specs=[pl.BlockSpec((_BR, 128), lambda i: (i, 0))] * 2,
        out_shape=[jax.ShapeDtypeStruct((_NP, 128), _f32)] * 2,
    )(t2, dinv_col, a2, c2)


def _tc_final(t3, z2, a3, c3, oh, Wfc, bfc):
    def body(t_ref, z_ref, a_ref, c_ref, oh_ref, w_ref, b_ref, o_ref,
             pool_acc, cnt_acc):
        i = pl.program_id(0)

        @pl.when(i == 0)
        def _():
            pool_acc[...] = jnp.zeros((_G, 128), _f32)
            cnt_acc[...] = jnp.zeros((_G, 128), _f32)

        h = jnp.maximum(t_ref[...] * a_ref[...] + c_ref[...], 0.0) + z_ref[...]
        ohm = oh_ref[...]
        dn = (((0,), (0,)), ((), ()))
        pool_acc[...] += lax.dot_general(ohm, h, dn,
                                         preferred_element_type=_f32)
        cnt_acc[...] += lax.dot_general(ohm, jnp.ones((_BR, 128), _f32), dn,
                                        preferred_element_type=_f32)

        @pl.when(i == _NB - 1)
        def _():
            pooled = pool_acc[...] / jnp.maximum(cnt_acc[...], 1.0)
            o_ref[...] = jnp.dot(pooled, w_ref[...],
                                 preferred_element_type=_f32) + b_ref[...]

    return pl.pallas_call(
        body,
        grid=(_NB,),
        in_specs=[
            pl.BlockSpec((_BR, 128), lambda i: (i, 0)),
            pl.BlockSpec((_BR, 128), lambda i: (i, 0)),
            pl.BlockSpec((1, 128), lambda i: (0, 0)),
            pl.BlockSpec((1, 128), lambda i: (0, 0)),
            pl.BlockSpec((_BR, _G), lambda i: (i, 0)),
            pl.BlockSpec((128, 1), lambda i: (0, 0)),
            pl.BlockSpec((1, 1), lambda i: (0, 0)),
        ],
        out_specs=pl.BlockSpec((_G, 1), lambda i: (0, 0)),
        out_shape=jax.ShapeDtypeStruct((_G, 1), _f32),
        scratch_shapes=[pltpu.VMEM((_G, 128), _f32),
                        pltpu.VMEM((_G, 128), _f32)],
    )(t3, z2, a3, c3, oh, Wfc, bfc)


def kernel(x, edge_index, edge_attr, batch,
           W1, b1, g1, be1, W2, b2, g2, be2, W3, b3, g3, be3, Wfc, bfc):
    src = edge_index[0].astype(_i32)
    dst = edge_index[1].astype(_i32)
    ew = edge_attr.astype(_f32)
    z1d = jnp.zeros((_CH,), _f32)
    degp = _sc_deg(dst, ew, z1d)
    xpad = jnp.pad(x[:, 0].astype(_f32), (0, _NP - _N))
    dinv2d, xp2d = _tc_prep(degp.reshape(2, _R2, 128), xpad.reshape(_R2, 128))
    sxp = _sc_sx(src, dst, ew, xp2d.reshape(_NP), z1d)
    z16 = jnp.zeros((_CH, 16), _f32)
    s2d, smu, svar = _tc_sstats(sxp.reshape(2, _R2, 128), xp2d, dinv2d)
    mu, var = smu[0, 0], svar[0, 0]
    a1 = (W1[0] * lax.rsqrt(var * W1[0] ** 2 + 1e-5) * g1)[None, :]
    c1 = (be1 - mu * a1[0])[None, :]
    dinv_col = dinv2d.reshape(_NP, 1)
    z1p = _tc_affine(s2d.reshape(_NP, 1), dinv_col, a1, c1)
    pe = _EPAD - _E
    src2 = jnp.pad(src, (0, pe)).reshape(_EPAD // 128, 128)
    dst2 = jnp.pad(dst, (0, pe)).reshape(_EPAD // 128, 128)
    ewp = jnp.pad(ew, (0, pe))
    agg1 = _sc_agg(z1p.reshape(4 * _NP, 16), src2, dst2, ewp, 4, z16)
    agg1t = agg1.transpose(1, 0, 2).reshape(_NP, 64)
    t2, ps2, pq2 = _tc_mm_stats(agg1t, z1p, dinv_col, W2, b2[None, :])
    a2, c2 = _tc_bnparams(ps2, pq2, g2[None, :], be2[None, :])
    z2, z2p = _tc_bnrelu(t2, dinv_col, a2, c2)

    agg2 = _sc_agg(z2p.reshape(8 * _NP, 16), src2, dst2, ewp, 8, z16)
    agg2t = agg2.transpose(1, 0, 2).reshape(_NP, 128)
    t3, ps3, pq3 = _tc_mm_stats(agg2t, z2p, dinv_col, W3, b3[None, :])
    a3, c3 = _tc_bnparams(ps3, pq3, g3[None, :], be3[None, :])

    batchp = jnp.pad(batch.astype(_i32), (0, _NP - _N), constant_values=_G)
    oh = (batchp[:, None] == jnp.arange(_G, dtype=_i32)[None, :]).astype(_f32)
    return _tc_final(t3, z2, a3, c3, oh, Wfc, bfc.reshape(1, 1))


# speedup vs baseline: 7.3754x; 1.3714x over previous
import dataclasses
import functools

import jax
import jax.numpy as jnp
from jax import lax
from jax.experimental import pallas as pl
from jax.experimental.pallas import tpu as pltpu
from jax.experimental.pallas import tpu_sc as plsc

_N = 100000
_E = 1600000
_G = 64
_NP = 100352
_NB = 98
_BR = 1024
_R2 = _NP // 128
_NSUB = 16
_CH = _NP // _NSUB
_EPADW = 100352
_EPAD = _EPADW * _NSUB

_f32 = jnp.float32
_i32 = jnp.int32


def _sc_mesh():
    return plsc.VectorSubcoreMesh(core_axis_name="c", subcore_axis_name="s")


def _sc_params():
    cp = pltpu.CompilerParams()
    fields = pltpu.CompilerParams.__dataclass_fields__
    if "needs_layout_passes" in fields:
        cp = dataclasses.replace(cp, needs_layout_passes=False)
    if "use_tc_tiling_on_sc" in fields:
        cp = dataclasses.replace(cp, use_tc_tiling_on_sc=False)
    return cp


def _sc_deg(dst, ew, zrow):
    WE = 5000
    EPW = _E // 32
    NWIN = EPW // WE

    @functools.partial(
        pl.kernel,
        out_type=jax.ShapeDtypeStruct((2, _NP), _f32),
        mesh=_sc_mesh(),
        compiler_params=_sc_params(),
        scratch_types=[
            pltpu.VMEM((WE,), _i32),
            pltpu.VMEM((WE,), _f32),
            pltpu.VMEM_SHARED((_NP,), _f32),
        ],
    )
    def k(dst_hbm, ew_hbm, z_hbm, out_hbm, dst_v, ew_v, acc_sh):
        cid = lax.axis_index("c")
        sid = lax.axis_index("s")
        pltpu.sync_copy(z_hbm, acc_sh.at[pl.ds(sid * _CH, _CH)])
        plsc.subcore_barrier()
        base = (cid * _NSUB + sid) * EPW

        @pl.loop(0, NWIN)
        def _(w):
            eb = base + w * WE
            pltpu.sync_copy(dst_hbm.at[pl.ds(eb, WE)], dst_v)
            pltpu.sync_copy(ew_hbm.at[pl.ds(eb, WE)], ew_v)
            pltpu.sync_copy(ew_v, acc_sh.at[dst_v], add=True)

        plsc.subcore_barrier()
        for c in range(2):
            @pl.when(cid == c)
            def _(c=c):
                pltpu.sync_copy(acc_sh.at[pl.ds(sid * _CH, _CH)],
                                out_hbm.at[c].at[pl.ds(sid * _CH, _CH)])

    return k(dst, ew, zrow)


def _sc_sx(src, dst, ew, xp, zrow):
    WE = 5000
    EPW = _E // 32
    NWIN = EPW // WE

    @functools.partial(
        pl.kernel,
        out_type=jax.ShapeDtypeStruct((2, _NP), _f32),
        mesh=_sc_mesh(),
        compiler_params=_sc_params(),
        scratch_types=[
            pltpu.VMEM((WE,), _i32),
            pltpu.VMEM((WE,), _i32),
            pltpu.VMEM((WE,), _f32),
            pltpu.VMEM((WE,), _f32),
            pltpu.VMEM((_NP,), _f32),
            pltpu.VMEM_SHARED((_NP,), _f32),
        ],
    )
    def k(src_hbm, dst_hbm, ew_hbm, xp_hbm, z_hbm, out_hbm,
          src_v, dst_v, ew_v, vals_v, xp_v, acc_sh):
        cid = lax.axis_index("c")
        sid = lax.axis_index("s")
        pltpu.sync_copy(z_hbm, acc_sh.at[pl.ds(sid * _CH, _CH)])
        pltpu.sync_copy(xp_hbm, xp_v)
        plsc.subcore_barrier()
        base = (cid * _NSUB + sid) * EPW

        @pl.loop(0, NWIN)
        def _(w):
            eb = base + w * WE
            pltpu.sync_copy(src_hbm.at[pl.ds(eb, WE)], src_v)
            pltpu.sync_copy(dst_hbm.at[pl.ds(eb, WE)], dst_v)
            pltpu.sync_copy(ew_hbm.at[pl.ds(eb, WE)], ew_v)

            @pl.loop(0, WE, step=16)
            def _(j):
                s16 = src_v[pl.ds(j, 16)]
                v16 = plsc.load_gather(xp_v, [s16])
                vals_v[pl.ds(j, 16)] = v16 * ew_v[pl.ds(j, 16)]

            pltpu.sync_copy(vals_v, acc_sh.at[dst_v], add=True)

        plsc.subcore_barrier()
        for c in range(2):
            @pl.when(cid == c)
            def _(c=c):
                pltpu.sync_copy(acc_sh.at[pl.ds(sid * _CH, _CH)],
                                out_hbm.at[c].at[pl.ds(sid * _CH, _CH)])

    return k(src, dst, ew, xp, zrow)


def _sc_agg(table, src, dst, ew, nslab, zrow16):
    WE = 512
    NCH = 4
    EPW = _EPADW
    NWIN = EPW // WE
    RB = EPW // 128

    buf = lambda: [pltpu.VMEM((NCH, 128), _i32),
                   pltpu.VMEM((NCH, 128), _i32),
                   pltpu.VMEM((NCH, 128), _i32),
                   pltpu.VMEM((WE,), _f32),
                   pltpu.VMEM((WE, 16), _f32)]

    @functools.partial(
        pl.kernel,
        out_type=jax.ShapeDtypeStruct((nslab, _NP, 16), _f32),
        mesh=_sc_mesh(),
        compiler_params=_sc_params(),
        scratch_types=buf() + buf() + [
            pltpu.VMEM_SHARED((_NP, 16), _f32),
            pltpu.SemaphoreType.DMA,
            pltpu.SemaphoreType.DMA,
        ],
    )
    def k(tab_hbm, src2_hbm, dst2_hbm, ew_hbm, z_hbm, out_hbm,
          src0, idx0, dst0, ew0, rows0, src1, idx1, dst1, ew1, rows1,
          acc_sh, sem0, sem1):
        cid = lax.axis_index("c")
        sid = lax.axis_index("s")
        bufs = ((src0, idx0, dst0, ew0, rows0, sem0),
                (src1, idx1, dst1, ew1, rows1, sem1))

        def issue(w, b, s):
            src_v, idx_v, dst_v, ew_v, rows_v, sem = bufs[b]
            rb = sid * RB + w * NCH
            pltpu.sync_copy(src2_hbm.at[pl.ds(rb, NCH)], src_v)
            pltpu.sync_copy(dst2_hbm.at[pl.ds(rb, NCH)], dst_v)
            pltpu.sync_copy(ew_hbm.at[pl.ds(sid * EPW + w * WE, WE)], ew_v)
            for g in range(NCH):
                @pl.loop(0, 128, step=16)
                def _(j, g=g):
                    s16 = src_v[g, pl.ds(j, 16)]
                    idx_v[g, pl.ds(j, 16)] = s16 * nslab + s
            for g in range(NCH):
                pltpu.async_copy(tab_hbm.at[idx_v.at[g]],
                                 rows_v.at[pl.ds(g * 128, 128)], sem)

        def consume(b):
            src_v, idx_v, dst_v, ew_v, rows_v, sem = bufs[b]
            for g in range(NCH):
                pltpu.make_async_copy(tab_hbm.at[idx_v.at[g]],
                                      rows_v.at[pl.ds(g * 128, 128)],
                                      sem).wait()

            @pl.loop(0, WE, step=4)
            def _(j):
                for u in range(4):
                    bc = plsc.load_gather(
                        ew_v, [jnp.full((16,), j + u, _i32)])
                    rows_v[j + u] = rows_v[j + u] * bc

            for g in range(NCH):
                pltpu.sync_copy(rows_v.at[pl.ds(g * 128, 128)],
                                acc_sh.at[dst_v.at[g]], add=True)

        for k in range(nslab // 2):
            s = k * 2 + cid
            pltpu.sync_copy(z_hbm, acc_sh.at[pl.ds(sid * _CH, _CH)])
            plsc.subcore_barrier()

            issue(0, 0, s)

            @pl.loop(0, NWIN, step=2)
            def _(w, s=s):
                issue(w + 1, 1, s)
                consume(0)

                @pl.when(w + 2 < NWIN)
                def _():
                    issue(w + 2, 0, s)

                consume(1)

            plsc.subcore_barrier()
            pltpu.sync_copy(acc_sh.at[pl.ds(sid * _CH, _CH)],
                            out_hbm.at[s].at[pl.ds(sid * _CH, _CH)])
            plsc.subcore_barrier()

    return k(table, src, dst, ew, zrow16)


def _tc_prep(degp, x2d):
    def body(d_ref, x_ref, dinv_ref, xp_ref):
        deg = d_ref[0] + d_ref[1] + 1.0
        dinv = lax.rsqrt(deg)
        dinv_ref[...] = dinv
        xp_ref[...] = x_ref[...] * dinv

    return pl.pallas_call(
        body,
        out_shape=[jax.ShapeDtypeStruct((_R2, 128), _f32)] * 2,
    )(degp, x2d)


def _tc_sstats(sxp, xp2d, dinv2d):
    def body(p_ref, xp_ref, di_ref, s_ref, mu_ref, var_ref):
        sarr = di_ref[...] * (p_ref[0] + p_ref[1] + xp_ref[...])
        s_ref[...] = sarr
        mu = jnp.sum(sarr) / _N
        var = jnp.sum(sarr * sarr) / _N - mu * mu
        mu_ref[...] = jnp.full((8, 128), mu)
        var_ref[...] = jnp.full((8, 128), var)

    return pl.pallas_call(
        body,
        out_shape=[jax.ShapeDtypeStruct((_R2, 128), _f32),
                   jax.ShapeDtypeStruct((8, 128), _f32),
                   jax.ShapeDtypeStruct((8, 128), _f32)],
    )(sxp, xp2d, dinv2d)


def _tc_affine(s_col, dinv_col, a1, c1):
    def body(s_ref, d_ref, a_ref, c_ref, o_ref):
        o_ref[...] = d_ref[...] * jnp.maximum(
            s_ref[...] * a_ref[...] + c_ref[...], 0.0)

    return pl.pallas_call(
        body,
        grid=(_NB,),
        in_specs=[
            pl.BlockSpec((_BR, 1), lambda i: (i, 0)),
            pl.BlockSpec((_BR, 1), lambda i: (i, 0)),
            pl.BlockSpec((1, 64), lambda i: (0, 0)),
            pl.BlockSpec((1, 64), lambda i: (0, 0)),
        ],
        out_specs=pl.BlockSpec((_BR, 64), lambda i: (i, 0)),
        out_shape=jax.ShapeDtypeStruct((_NP, 64), _f32),
    )(s_col, dinv_col, a1, c1)


def _tc_mm_stats(aggt, zp, dinv_col, Wmat, brow):
    din, dout = Wmat.shape

    def body(agg_ref, zp_ref, d_ref, w_ref, b_ref, t_ref, ps_ref, pq_ref):
        i = pl.program_id(0)
        u = d_ref[...] * (agg_ref[...] + zp_ref[...])
        t = jnp.dot(u, w_ref[...], preferred_element_type=_f32) + b_ref[...]
        t_ref[...] = t
        rid = i * _BR + lax.broadcasted_iota(_i32, (_BR, 1), 0)
        tm = t * (rid < _N).astype(_f32)
        ps_ref[...] = jnp.sum(tm, axis=0, keepdims=True)[None]
        pq_ref[...] = jnp.sum(tm * tm, axis=0, keepdims=True)[None]

    return pl.pallas_call(
        body,
        grid=(_NB,),
        in_specs=[
            pl.BlockSpec((_BR, din), lambda i: (i, 0)),
            pl.BlockSpec((_BR, din), lambda i: (i, 0)),
            pl.BlockSpec((_BR, 1), lambda i: (i, 0)),
            pl.BlockSpec((din, dout), lambda i: (0, 0)),
            pl.BlockSpec((1, dout), lambda i: (0, 0)),
        ],
        out_specs=[
            pl.BlockSpec((_BR, dout), lambda i: (i, 0)),
            pl.BlockSpec((1, 1, dout), lambda i: (i, 0, 0)),
            pl.BlockSpec((1, 1, dout), lambda i: (i, 0, 0)),
        ],
        out_shape=[jax.ShapeDtypeStruct((_NP, dout), _f32),
                   jax.ShapeDtypeStruct((_NB, 1, dout), _f32),
                   jax.ShapeDtypeStruct((_NB, 1, dout), _f32)],
    )(aggt, zp, dinv_col, Wmat, brow)


def _tc_bnparams(ps, pq, grow, berow):
    dout = ps.shape[-1]

    def body(ps_ref, pq_ref, g_ref, b_ref, a_ref, c_ref):
        ssum = jnp.sum(ps_ref[...], axis=(0, 1))
        sq = jnp.sum(pq_ref[...], axis=(0, 1))
        mu = ssum / _N
        var = sq / _N - mu * mu
        a = lax.rsqrt(var + 1e-5) * g_ref[0]
        c = b_ref[0] - mu * a
        a_ref[...] = a[None, :]
        c_ref[...] = c[None, :]

    return pl.pallas_call(
        body,
        out_shape=[jax.ShapeDtypeStruct((1, dout), _f32)] * 2,
    )(ps, pq, grow, berow)


def _tc_bnrelu(t2, dinv_col, a2, c2):
    def body(t_ref, d_ref, a_ref, c_ref, z_ref, zp_ref):
        z = jnp.maximum(t_ref[...] * a_ref[...] + c_ref[...], 0.0)
        z_ref[...] = z
        zp_ref[...] = d_ref[...] * z

    return pl.pallas_call(
        body,
        grid=(_NB,),
        in_specs=[
            pl.BlockSpec((_BR, 128), lambda i: (i, 0)),
            pl.BlockSpec((_BR, 1), lambda i: (i, 0)),
            pl.BlockSpec((1, 128), lambda i: (0, 0)),
            pl.BlockSpec((1, 128), lambda i: (0, 0)),
        ],
        out_specs=[pl.BlockSpec((_BR, 128), lambda i: (i, 0))] * 2,
        out_shape=[jax.ShapeDtypeStruct((_NP, 128), _f32)] * 2,
    )(t2, dinv_col, a2, c2)


def _tc_final(t3, z2, a3, c3, oh, Wfc, bfc):
    def body(t_ref, z_ref, a_ref, c_ref, oh_ref, w_ref, b_ref, o_ref,
             pool_acc, cnt_acc):
        i = pl.program_id(0)

        @pl.when(i == 0)
        def _():
            pool_acc[...] = jnp.zeros((_G, 128), _f32)
            cnt_acc[...] = jnp.zeros((_G, 128), _f32)

        h = jnp.maximum(t_ref[...] * a_ref[...] + c_ref[...], 0.0) + z_ref[...]
        ohm = oh_ref[...]
        dn = (((0,), (0,)), ((), ()))
        pool_acc[...] += lax.dot_general(ohm, h, dn,
                                         preferred_element_type=_f32)
        cnt_acc[...] += lax.dot_general(ohm, jnp.ones((_BR, 128), _f32), dn,
                                        preferred_element_type=_f32)

        @pl.when(i == _NB - 1)
        def _():
            pooled = pool_acc[...] / jnp.maximum(cnt_acc[...], 1.0)
            o_ref[...] = jnp.dot(pooled, w_ref[...],
                                 preferred_element_type=_f32) + b_ref[...]

    return pl.pallas_call(
        body,
        grid=(_NB,),
        in_specs=[
            pl.BlockSpec((_BR, 128), lambda i: (i, 0)),
            pl.BlockSpec((_BR, 128), lambda i: (i, 0)),
            pl.BlockSpec((1, 128), lambda i: (0, 0)),
            pl.BlockSpec((1, 128), lambda i: (0, 0)),
            pl.BlockSpec((_BR, _G), lambda i: (i, 0)),
            pl.BlockSpec((128, 1), lambda i: (0, 0)),
            pl.BlockSpec((1, 1), lambda i: (0, 0)),
        ],
        out_specs=pl.BlockSpec((_G, 1), lambda i: (0, 0)),
        out_shape=jax.ShapeDtypeStruct((_G, 1), _f32),
        scratch_shapes=[pltpu.VMEM((_G, 128), _f32),
                        pltpu.VMEM((_G, 128), _f32)],
    )(t3, z2, a3, c3, oh, Wfc, bfc)


def kernel(x, edge_index, edge_attr, batch,
           W1, b1, g1, be1, W2, b2, g2, be2, W3, b3, g3, be3, Wfc, bfc):
    src = edge_index[0].astype(_i32)
    dst = edge_index[1].astype(_i32)
    ew = edge_attr.astype(_f32)
    z1d = jnp.zeros((_CH,), _f32)
    degp = _sc_deg(dst, ew, z1d)
    xpad = jnp.pad(x[:, 0].astype(_f32), (0, _NP - _N))
    dinv2d, xp2d = _tc_prep(degp.reshape(2, _R2, 128), xpad.reshape(_R2, 128))
    sxp = _sc_sx(src, dst, ew, xp2d.reshape(_NP), z1d)
    z16 = jnp.zeros((_CH, 16), _f32)
    s2d, smu, svar = _tc_sstats(sxp.reshape(2, _R2, 128), xp2d, dinv2d)
    mu, var = smu[0, 0], svar[0, 0]
    a1 = (W1[0] * lax.rsqrt(var * W1[0] ** 2 + 1e-5) * g1)[None, :]
    c1 = (be1 - mu * a1[0])[None, :]
    dinv_col = dinv2d.reshape(_NP, 1)
    z1p = _tc_affine(s2d.reshape(_NP, 1), dinv_col, a1, c1)
    pe = _EPAD - _E
    src2 = jnp.pad(src, (0, pe)).reshape(_EPAD // 128, 128)
    dst2 = jnp.pad(dst, (0, pe)).reshape(_EPAD // 128, 128)
    ewp = jnp.pad(ew, (0, pe))
    agg1 = _sc_agg(z1p.reshape(4 * _NP, 16), src2, dst2, ewp, 4, z16)
    agg1t = agg1.transpose(1, 0, 2).reshape(_NP, 64)
    t2, ps2, pq2 = _tc_mm_stats(agg1t, z1p, dinv_col, W2, b2[None, :])
    a2, c2 = _tc_bnparams(ps2, pq2, g2[None, :], be2[None, :])
    z2, z2p = _tc_bnrelu(t2, dinv_col, a2, c2)

    agg2 = _sc_agg(z2p.reshape(8 * _NP, 16), src2, dst2, ewp, 8, z16)
    agg2t = agg2.transpose(1, 0, 2).reshape(_NP, 128)
    t3, ps3, pq3 = _tc_mm_stats(agg2t, z2p, dinv_col, W3, b3[None, :])
    a3, c3 = _tc_bnparams(ps3, pq3, g3[None, :], be3[None, :])

    batchp = jnp.pad(batch.astype(_i32), (0, _NP - _N), constant_values=_G)
    oh = (batchp[:, None] == jnp.arange(_G, dtype=_i32)[None, :]).astype(_f32)
    return _tc_final(t3, z2, a3, c3, oh, Wfc, bfc.reshape(1, 1))


# speedup vs baseline: 11.3970x; 1.5453x over previous
import dataclasses
import functools

import jax
import jax.numpy as jnp
from jax import lax
from jax.experimental import pallas as pl
from jax.experimental.pallas import tpu as pltpu
from jax.experimental.pallas import tpu_sc as plsc

_N = 100000
_E = 1600000
_G = 64
_NP = 100352
_NB = 98
_BR = 1024
_R2 = _NP // 128
_NSUB = 16
_CH = _NP // _NSUB
_EPADW = 100352
_EPAD = _EPADW * _NSUB

_f32 = jnp.float32
_i32 = jnp.int32


def _sc_mesh():
    return plsc.VectorSubcoreMesh(core_axis_name="c", subcore_axis_name="s")


def _sc_params():
    cp = pltpu.CompilerParams()
    fields = pltpu.CompilerParams.__dataclass_fields__
    if "needs_layout_passes" in fields:
        cp = dataclasses.replace(cp, needs_layout_passes=False)
    if "use_tc_tiling_on_sc" in fields:
        cp = dataclasses.replace(cp, use_tc_tiling_on_sc=False)
    return cp


def _sc_deg(dst, ew, zrow):
    WE = 5000
    EPW = _E // 32
    NWIN = EPW // WE

    @functools.partial(
        pl.kernel,
        out_type=jax.ShapeDtypeStruct((2, _NP), _f32),
        mesh=_sc_mesh(),
        compiler_params=_sc_params(),
        scratch_types=[
            pltpu.VMEM((WE,), _i32),
            pltpu.VMEM((WE,), _f32),
            pltpu.VMEM_SHARED((_NP,), _f32),
        ],
    )
    def k(dst_hbm, ew_hbm, z_hbm, out_hbm, dst_v, ew_v, acc_sh):
        cid = lax.axis_index("c")
        sid = lax.axis_index("s")
        pltpu.sync_copy(z_hbm, acc_sh.at[pl.ds(sid * _CH, _CH)])
        plsc.subcore_barrier()
        base = (cid * _NSUB + sid) * EPW

        @pl.loop(0, NWIN)
        def _(w):
            eb = base + w * WE
            pltpu.sync_copy(dst_hbm.at[pl.ds(eb, WE)], dst_v)
            pltpu.sync_copy(ew_hbm.at[pl.ds(eb, WE)], ew_v)
            pltpu.sync_copy(ew_v, acc_sh.at[dst_v], add=True)

        plsc.subcore_barrier()
        for c in range(2):
            @pl.when(cid == c)
            def _(c=c):
                pltpu.sync_copy(acc_sh.at[pl.ds(sid * _CH, _CH)],
                                out_hbm.at[c].at[pl.ds(sid * _CH, _CH)])

    return k(dst, ew, zrow)


def _sc_sx(src, dst, ew, xp, zrow):
    WE = 5000
    EPW = _E // 32
    NWIN = EPW // WE

    @functools.partial(
        pl.kernel,
        out_type=jax.ShapeDtypeStruct((2, _NP), _f32),
        mesh=_sc_mesh(),
        compiler_params=_sc_params(),
        scratch_types=[
            pltpu.VMEM((WE,), _i32),
            pltpu.VMEM((WE,), _i32),
            pltpu.VMEM((WE,), _f32),
            pltpu.VMEM((WE,), _f32),
            pltpu.VMEM((_NP,), _f32),
            pltpu.VMEM_SHARED((_NP,), _f32),
        ],
    )
    def k(src_hbm, dst_hbm, ew_hbm, xp_hbm, z_hbm, out_hbm,
          src_v, dst_v, ew_v, vals_v, xp_v, acc_sh):
        cid = lax.axis_index("c")
        sid = lax.axis_index("s")
        pltpu.sync_copy(z_hbm, acc_sh.at[pl.ds(sid * _CH, _CH)])
        pltpu.sync_copy(xp_hbm, xp_v)
        plsc.subcore_barrier()
        base = (cid * _NSUB + sid) * EPW

        @pl.loop(0, NWIN)
        def _(w):
            eb = base + w * WE
            pltpu.sync_copy(src_hbm.at[pl.ds(eb, WE)], src_v)
            pltpu.sync_copy(dst_hbm.at[pl.ds(eb, WE)], dst_v)
            pltpu.sync_copy(ew_hbm.at[pl.ds(eb, WE)], ew_v)

            @pl.loop(0, WE, step=16)
            def _(j):
                s16 = src_v[pl.ds(j, 16)]
                v16 = plsc.load_gather(xp_v, [s16])
                vals_v[pl.ds(j, 16)] = v16 * ew_v[pl.ds(j, 16)]

            pltpu.sync_copy(vals_v, acc_sh.at[dst_v], add=True)

        plsc.subcore_barrier()
        for c in range(2):
            @pl.when(cid == c)
            def _(c=c):
                pltpu.sync_copy(acc_sh.at[pl.ds(sid * _CH, _CH)],
                                out_hbm.at[c].at[pl.ds(sid * _CH, _CH)])

    return k(src, dst, ew, xp, zrow)


def _sc_agg(table, src, dst, ew, nslab, zrow16):
    WE = 512
    NCH = 4
    EPW = _EPADW
    NWIN = EPW // WE
    RB = EPW // 128

    buf = lambda: [pltpu.VMEM((NCH, 128), _i32),
                   pltpu.VMEM((NCH, 128), _i32),
                   pltpu.VMEM((NCH, 128), _i32),
                   pltpu.VMEM((WE,), _f32),
                   pltpu.VMEM((WE, 16), _f32)]

    @functools.partial(
        pl.kernel,
        out_type=jax.ShapeDtypeStruct((nslab, _NP, 16), _f32),
        mesh=_sc_mesh(),
        compiler_params=_sc_params(),
        scratch_types=buf() + buf() + [
            pltpu.VMEM_SHARED((_NP, 16), _f32),
            pltpu.SemaphoreType.DMA,
            pltpu.SemaphoreType.DMA,
        ],
    )
    def k(tab_hbm, src2_hbm, dst2_hbm, ew_hbm, z_hbm, out_hbm,
          src0, idx0, dst0, ew0, rows0, src1, idx1, dst1, ew1, rows1,
          acc_sh, sem0, sem1):
        cid = lax.axis_index("c")
        sid = lax.axis_index("s")
        bufs = ((src0, idx0, dst0, ew0, rows0, sem0),
                (src1, idx1, dst1, ew1, rows1, sem1))

        def issue(w, b, s):
            src_v, idx_v, dst_v, ew_v, rows_v, sem = bufs[b]
            rb = sid * RB + w * NCH
            pltpu.sync_copy(src2_hbm.at[pl.ds(rb, NCH)], src_v)
            pltpu.sync_copy(dst2_hbm.at[pl.ds(rb, NCH)], dst_v)
            pltpu.sync_copy(ew_hbm.at[pl.ds(sid * EPW + w * WE, WE)], ew_v)
            for g in range(NCH):
                @pl.loop(0, 128, step=16)
                def _(j, g=g):
                    s16 = src_v[g, pl.ds(j, 16)]
                    idx_v[g, pl.ds(j, 16)] = s16 * nslab + s
            for g in range(NCH):
                pltpu.async_copy(tab_hbm.at[idx_v.at[g]],
                                 rows_v.at[pl.ds(g * 128, 128)], sem)

        def consume(b):
            src_v, idx_v, dst_v, ew_v, rows_v, sem = bufs[b]
            for g in range(NCH):
                pltpu.make_async_copy(tab_hbm.at[idx_v.at[g]],
                                      rows_v.at[pl.ds(g * 128, 128)],
                                      sem).wait()

            dn = lax.GatherDimensionNumbers(offset_dims=(),
                                            collapsed_slice_dims=(0,),
                                            start_index_map=(0,))

            @pl.loop(0, WE, step=16)
            def _(j):
                ew16 = ew_v[pl.ds(j, 16)]
                for u in range(16):
                    bc = lax.gather(
                        ew16, jnp.full((16, 1), u, _i32), dn,
                        slice_sizes=(1,),
                        mode=lax.GatherScatterMode.PROMISE_IN_BOUNDS)
                    rows_v[j + u] = rows_v[j + u] * bc

            for g in range(NCH):
                pltpu.sync_copy(rows_v.at[pl.ds(g * 128, 128)],
                                acc_sh.at[dst_v.at[g]], add=True)

        for k in range(nslab // 2):
            s = k * 2 + cid
            pltpu.sync_copy(z_hbm, acc_sh.at[pl.ds(sid * _CH, _CH)])
            plsc.subcore_barrier()

            issue(0, 0, s)

            @pl.loop(0, NWIN, step=2)
            def _(w, s=s):
                issue(w + 1, 1, s)
                consume(0)

                @pl.when(w + 2 < NWIN)
                def _():
                    issue(w + 2, 0, s)

                consume(1)

            plsc.subcore_barrier()
            pltpu.sync_copy(acc_sh.at[pl.ds(sid * _CH, _CH)],
                            out_hbm.at[s].at[pl.ds(sid * _CH, _CH)])
            plsc.subcore_barrier()

    return k(table, src, dst, ew, zrow16)


def _tc_prep(degp, x2d):
    def body(d_ref, x_ref, dinv_ref, xp_ref):
        deg = d_ref[0] + d_ref[1] + 1.0
        dinv = lax.rsqrt(deg)
        dinv_ref[...] = dinv
        xp_ref[...] = x_ref[...] * dinv

    return pl.pallas_call(
        body,
        out_shape=[jax.ShapeDtypeStruct((_R2, 128), _f32)] * 2,
    )(degp, x2d)


def _tc_sstats(sxp, xp2d, dinv2d):
    def body(p_ref, xp_ref, di_ref, s_ref, mu_ref, var_ref):
        sarr = di_ref[...] * (p_ref[0] + p_ref[1] + xp_ref[...])
        s_ref[...] = sarr
        mu = jnp.sum(sarr) / _N
        var = jnp.sum(sarr * sarr) / _N - mu * mu
        mu_ref[...] = jnp.full((8, 128), mu)
        var_ref[...] = jnp.full((8, 128), var)

    return pl.pallas_call(
        body,
        out_shape=[jax.ShapeDtypeStruct((_R2, 128), _f32),
                   jax.ShapeDtypeStruct((8, 128), _f32),
                   jax.ShapeDtypeStruct((8, 128), _f32)],
    )(sxp, xp2d, dinv2d)


def _tc_affine(s_col, dinv_col, a1, c1):
    def body(s_ref, d_ref, a_ref, c_ref, o_ref):
        o_ref[...] = d_ref[...] * jnp.maximum(
            s_ref[...] * a_ref[...] + c_ref[...], 0.0)

    return pl.pallas_call(
        body,
        grid=(_NB,),
        in_specs=[
            pl.BlockSpec((_BR, 1), lambda i: (i, 0)),
            pl.BlockSpec((_BR, 1), lambda i: (i, 0)),
            pl.BlockSpec((1, 64), lambda i: (0, 0)),
            pl.BlockSpec((1, 64), lambda i: (0, 0)),
        ],
        out_specs=pl.BlockSpec((_BR, 64), lambda i: (i, 0)),
        out_shape=jax.ShapeDtypeStruct((_NP, 64), _f32),
    )(s_col, dinv_col, a1, c1)


def _tc_mm_stats(aggt, zp, dinv_col, Wmat, brow):
    din, dout = Wmat.shape

    def body(agg_ref, zp_ref, d_ref, w_ref, b_ref, t_ref, ps_ref, pq_ref):
        i = pl.program_id(0)
        u = d_ref[...] * (agg_ref[...] + zp_ref[...])
        t = jnp.dot(u, w_ref[...], preferred_element_type=_f32) + b_ref[...]
        t_ref[...] = t
        rid = i * _BR + lax.broadcasted_iota(_i32, (_BR, 1), 0)
        tm = t * (rid < _N).astype(_f32)
        ps_ref[...] = jnp.sum(tm, axis=0, keepdims=True)[None]
        pq_ref[...] = jnp.sum(tm * tm, axis=0, keepdims=True)[None]

    return pl.pallas_call(
        body,
        grid=(_NB,),
        in_specs=[
            pl.BlockSpec((_BR, din), lambda i: (i, 0)),
            pl.BlockSpec((_BR, din), lambda i: (i, 0)),
            pl.BlockSpec((_BR, 1), lambda i: (i, 0)),
            pl.BlockSpec((din, dout), lambda i: (0, 0)),
            pl.BlockSpec((1, dout), lambda i: (0, 0)),
        ],
        out_specs=[
            pl.BlockSpec((_BR, dout), lambda i: (i, 0)),
            pl.BlockSpec((1, 1, dout), lambda i: (i, 0, 0)),
            pl.BlockSpec((1, 1, dout), lambda i: (i, 0, 0)),
        ],
        out_shape=[jax.ShapeDtypeStruct((_NP, dout), _f32),
                   jax.ShapeDtypeStruct((_NB, 1, dout), _f32),
                   jax.ShapeDtypeStruct((_NB, 1, dout), _f32)],
    )(aggt, zp, dinv_col, Wmat, brow)


def _tc_bnparams(ps, pq, grow, berow):
    dout = ps.shape[-1]

    def body(ps_ref, pq_ref, g_ref, b_ref, a_ref, c_ref):
        ssum = jnp.sum(ps_ref[...], axis=(0, 1))
        sq = jnp.sum(pq_ref[...], axis=(0, 1))
        mu = ssum / _N
        var = sq / _N - mu * mu
        a = lax.rsqrt(var + 1e-5) * g_ref[0]
        c = b_ref[0] - mu * a
        a_ref[...] = a[None, :]
        c_ref[...] = c[None, :]

    return pl.pallas_call(
        body,
        out_shape=[jax.ShapeDtypeStruct((1, dout), _f32)] * 2,
    )(ps, pq, grow, berow)


def _tc_bnrelu(t2, dinv_col, a2, c2):
    def body(t_ref, d_ref, a_ref, c_ref, z_ref, zp_ref):
        z = jnp.maximum(t_ref[...] * a_ref[...] + c_ref[...], 0.0)
        z_ref[...] = z
        zp_ref[...] = d_ref[...] * z

    return pl.pallas_call(
        body,
        grid=(_NB,),
        in_specs=[
            pl.BlockSpec((_BR, 128), lambda i: (i, 0)),
            pl.BlockSpec((_BR, 1), lambda i: (i, 0)),
            pl.BlockSpec((1, 128), lambda i: (0, 0)),
            pl.BlockSpec((1, 128), lambda i: (0, 0)),
        ],
        out_specs=[pl.BlockSpec((_BR, 128), lambda i: (i, 0))] * 2,
        out_shape=[jax.ShapeDtypeStruct((_NP, 128), _f32)] * 2,
    )(t2, dinv_col, a2, c2)


def _tc_final(t3, z2, a3, c3, oh, Wfc, bfc):
    def body(t_ref, z_ref, a_ref, c_ref, oh_ref, w_ref, b_ref, o_ref,
             pool_acc, cnt_acc):
        i = pl.program_id(0)

        @pl.when(i == 0)
        def _():
            pool_acc[...] = jnp.zeros((_G, 128), _f32)
            cnt_acc[...] = jnp.zeros((_G, 128), _f32)

        h = jnp.maximum(t_ref[...] * a_ref[...] + c_ref[...], 0.0) + z_ref[...]
        ohm = oh_ref[...]
        dn = (((0,), (0,)), ((), ()))
        pool_acc[...] += lax.dot_general(ohm, h, dn,
                                         preferred_element_type=_f32)
        cnt_acc[...] += lax.dot_general(ohm, jnp.ones((_BR, 128), _f32), dn,
                                        preferred_element_type=_f32)

        @pl.when(i == _NB - 1)
        def _():
            pooled = pool_acc[...] / jnp.maximum(cnt_acc[...], 1.0)
            o_ref[...] = jnp.dot(pooled, w_ref[...],
                                 preferred_element_type=_f32) + b_ref[...]

    return pl.pallas_call(
        body,
        grid=(_NB,),
        in_specs=[
            pl.BlockSpec((_BR, 128), lambda i: (i, 0)),
            pl.BlockSpec((_BR, 128), lambda i: (i, 0)),
            pl.BlockSpec((1, 128), lambda i: (0, 0)),
            pl.BlockSpec((1, 128), lambda i: (0, 0)),
            pl.BlockSpec((_BR, _G), lambda i: (i, 0)),
            pl.BlockSpec((128, 1), lambda i: (0, 0)),
            pl.BlockSpec((1, 1), lambda i: (0, 0)),
        ],
        out_specs=pl.BlockSpec((_G, 1), lambda i: (0, 0)),
        out_shape=jax.ShapeDtypeStruct((_G, 1), _f32),
        scratch_shapes=[pltpu.VMEM((_G, 128), _f32),
                        pltpu.VMEM((_G, 128), _f32)],
    )(t3, z2, a3, c3, oh, Wfc, bfc)


def kernel(x, edge_index, edge_attr, batch,
           W1, b1, g1, be1, W2, b2, g2, be2, W3, b3, g3, be3, Wfc, bfc):
    src = edge_index[0].astype(_i32)
    dst = edge_index[1].astype(_i32)
    ew = edge_attr.astype(_f32)
    z1d = jnp.zeros((_CH,), _f32)
    degp = _sc_deg(dst, ew, z1d)
    xpad = jnp.pad(x[:, 0].astype(_f32), (0, _NP - _N))
    dinv2d, xp2d = _tc_prep(degp.reshape(2, _R2, 128), xpad.reshape(_R2, 128))
    sxp = _sc_sx(src, dst, ew, xp2d.reshape(_NP), z1d)
    z16 = jnp.zeros((_CH, 16), _f32)
    s2d, smu, svar = _tc_sstats(sxp.reshape(2, _R2, 128), xp2d, dinv2d)
    mu, var = smu[0, 0], svar[0, 0]
    a1 = (W1[0] * lax.rsqrt(var * W1[0] ** 2 + 1e-5) * g1)[None, :]
    c1 = (be1 - mu * a1[0])[None, :]
    dinv_col = dinv2d.reshape(_NP, 1)
    z1p = _tc_affine(s2d.reshape(_NP, 1), dinv_col, a1, c1)
    pe = _EPAD - _E
    src2 = jnp.pad(src, (0, pe)).reshape(_EPAD // 128, 128)
    dst2 = jnp.pad(dst, (0, pe)).reshape(_EPAD // 128, 128)
    ewp = jnp.pad(ew, (0, pe))
    agg1 = _sc_agg(z1p.reshape(4 * _NP, 16), src2, dst2, ewp, 4, z16)
    agg1t = agg1.transpose(1, 0, 2).reshape(_NP, 64)
    t2, ps2, pq2 = _tc_mm_stats(agg1t, z1p, dinv_col, W2, b2[None, :])
    a2, c2 = _tc_bnparams(ps2, pq2, g2[None, :], be2[None, :])
    z2, z2p = _tc_bnrelu(t2, dinv_col, a2, c2)

    agg2 = _sc_agg(z2p.reshape(8 * _NP, 16), src2, dst2, ewp, 8, z16)
    agg2t = agg2.transpose(1, 0, 2).reshape(_NP, 128)
    t3, ps3, pq3 = _tc_mm_stats(agg2t, z2p, dinv_col, W3, b3[None, :])
    a3, c3 = _tc_bnparams(ps3, pq3, g3[None, :], be3[None, :])

    batchp = jnp.pad(batch.astype(_i32), (0, _NP - _N), constant_values=_G)
    oh = (batchp[:, None] == jnp.arange(_G, dtype=_i32)[None, :]).astype(_f32)
    return _tc_final(t3, z2, a3, c3, oh, Wfc, bfc.reshape(1, 1))
